# Initial kernel scaffold; baseline (speedup 1.0000x reference)
import jax
import jax.numpy as jnp
from jax.experimental import pallas as pl


def kernel(vfeat, ffeat, aw1, ab1, aw2, ab2, aw3, ab3, bw1, bb1, bw2, bb2, bw3, bb3):
    raise NotImplementedError("write your pallas kernel here")


# speedup vs baseline: 96.0000x; 96.0000x over previous
import functools

import jax
import jax.numpy as jnp
from jax.experimental import pallas as pl

_B = 4
_C = 1024
_T = 2048
_H = 512
_JBLK = 256
_K = _T // 8
_ACTION_T = 0.55
_BACKGROUND_T = 0.45
_SIM_T = 0.05
_BS_ITERS = 32


def _attn_kernel(x_ref, w1_ref, b1_ref, w2_ref, b2_ref, w3_ref, b3_ref,
                 atn_ref, nrm_ref):
    x = x_ref[0]
    nrm = jnp.sqrt(jnp.sum(x * x, axis=0, keepdims=True))
    nrm_ref[0] = jnp.maximum(nrm, 1e-12)

    def conv3(h, w_ref, b_ref):
        y0 = jnp.dot(w_ref[0], h, preferred_element_type=jnp.float32)
        y1 = jnp.dot(w_ref[1], h, preferred_element_type=jnp.float32)
        y2 = jnp.dot(w_ref[2], h, preferred_element_type=jnp.float32)
        z = jnp.zeros((y0.shape[0], 1), jnp.float32)
        out = (y1
               + jnp.concatenate([z, y0[:, :-1]], axis=1)
               + jnp.concatenate([y2[:, 1:], z], axis=1)
               + b_ref[...])
        return jax.nn.leaky_relu(out, 0.2)

    h1 = conv3(x, w1_ref, b1_ref)
    h2 = conv3(h1, w2_ref, b2_ref)
    h3 = jnp.dot(w3_ref[...], h2, preferred_element_type=jnp.float32)
    atn_ref[0] = jax.nn.sigmoid(h3 + b3_ref[...])


def _attention(x, w1, b1, w2, b2, w3, b3):
    w1t = jnp.transpose(w1, (2, 0, 1))
    w2t = jnp.transpose(w2, (2, 0, 1))
    b1c = b1[:, None]
    b2c = b2[:, None]
    w3r = w3[:, :, 0]
    b3r = b3[None, :]
    full = lambda a: pl.BlockSpec(a.shape, lambda b: (0,) * a.ndim)
    atn, nrm = pl.pallas_call(
        _attn_kernel,
        grid=(_B,),
        in_specs=[
            pl.BlockSpec((1, _C, _T), lambda b: (b, 0, 0)),
            full(w1t), full(b1c), full(w2t), full(b2c), full(w3r), full(b3r),
        ],
        out_specs=(
            pl.BlockSpec((1, 1, _T), lambda b: (b, 0, 0)),
            pl.BlockSpec((1, 1, _T), lambda b: (b, 0, 0)),
        ),
        out_shape=(
            jax.ShapeDtypeStruct((_B, 1, _T), jnp.float32),
            jax.ShapeDtypeStruct((_B, 1, _T), jnp.float32),
        ),
    )(x, w1t, b1c, w2t, b2c, w3r, b3r)
    return atn, nrm


def _core_kernel(vf_ref, ff_ref, nv_ref, nf_ref, av_ref, af_ref,
                 outv_ref, outf_ref):
    j = pl.program_id(1)
    js = j * _JBLK
    vf = vf_ref[0]
    ff = ff_ref[0]
    vfj = vf_ref[0, :, pl.ds(js, _JBLK)]
    ffj = ff_ref[0, :, pl.ds(js, _JBLK)]

    cdims = (((0,), (0,)), ((), ()))
    vsim = jax.lax.dot_general(vfj, vf, cdims,
                               preferred_element_type=jnp.float32)
    fsim = jax.lax.dot_general(ffj, ff, cdims,
                               preferred_element_type=jnp.float32)
    inv_nv = 1.0 / nv_ref[0]
    inv_nf = 1.0 / nf_ref[0]
    inv_nv_c = jnp.reshape(1.0 / nv_ref[0, :, pl.ds(js, _JBLK)], (_JBLK, 1))
    inv_nf_c = jnp.reshape(1.0 / nf_ref[0, :, pl.ds(js, _JBLK)], (_JBLK, 1))
    fus = 0.5 * (vsim * inv_nv_c * inv_nv + fsim * inv_nf_c * inv_nf)
    fus = jnp.where(fus < _SIM_T, 0.0, fus)

    def bs_step(_, lohi):
        lo, hi = lohi
        mid = 0.5 * (lo + hi)
        cnt = jnp.sum((fus >= mid).astype(jnp.float32), axis=1,
                      keepdims=True)
        pred = cnt >= _K
        return jnp.where(pred, mid, lo), jnp.where(pred, hi, mid)

    lo, _ = jax.lax.fori_loop(
        0, _BS_ITERS, bs_step,
        (jnp.zeros((_JBLK, 1), jnp.float32),
         jnp.full((_JBLK, 1), 2.0, jnp.float32)))
    keep = (fus >= lo).astype(jnp.float32)

    av = av_ref[0]
    af = af_ref[0]
    act = ((av >= _ACTION_T) & (af >= _ACTION_T)).astype(jnp.float32)
    bg = ((av < _BACKGROUND_T) & (af < _BACKGROUND_T)).astype(jnp.float32)
    ab = act + bg
    avj = av_ref[0, :, pl.ds(js, _JBLK)]
    afj = af_ref[0, :, pl.ds(js, _JBLK)]
    act_c = jnp.reshape(
        ((avj >= _ACTION_T) & (afj >= _ACTION_T)).astype(jnp.float32),
        (_JBLK, 1))
    bg_c = jnp.reshape(
        ((avj < _BACKGROUND_T) & (afj < _BACKGROUND_T)).astype(jnp.float32),
        (_JBLK, 1))
    amb_c = 1.0 - act_c - bg_c
    gi = jax.lax.broadcasted_iota(jnp.int32, (_JBLK, _T), 1)
    gj = js + jax.lax.broadcasted_iota(jnp.int32, (_JBLK, _T), 0)
    eye = (gi == gj).astype(jnp.float32)
    m = act_c * act + bg_c * bg + amb_c * jnp.maximum(ab, eye)

    masked = fus * keep * m
    den = jnp.sum(masked, axis=1, keepdims=True)
    adj_t = masked * (1.0 / jnp.maximum(den, 1e-12))

    rdims = (((1,), (1,)), ((), ()))
    avg_v = jax.lax.dot_general(vf, adj_t, rdims,
                                preferred_element_type=jnp.float32)
    avg_f = jax.lax.dot_general(ff, adj_t, rdims,
                                preferred_element_type=jnp.float32)
    outv_ref[0] = (vfj + avg_v) * 0.5
    outf_ref[0] = (ffj + avg_f) * 0.5


def _core(vfeat, ffeat, nv, nf, av, af):
    nj = _T // _JBLK
    feat_spec = pl.BlockSpec((1, _C, _T), lambda b, j: (b, 0, 0))
    vec_spec = pl.BlockSpec((1, 1, _T), lambda b, j: (b, 0, 0))
    out_spec = pl.BlockSpec((1, _C, _JBLK), lambda b, j: (b, 0, j))
    return pl.pallas_call(
        _core_kernel,
        grid=(_B, nj),
        in_specs=[feat_spec, feat_spec, vec_spec, vec_spec, vec_spec,
                  vec_spec],
        out_specs=(out_spec, out_spec),
        out_shape=(
            jax.ShapeDtypeStruct((_B, _C, _T), jnp.float32),
            jax.ShapeDtypeStruct((_B, _C, _T), jnp.float32),
        ),
    )(vfeat, ffeat, nv, nf, av, af)


@jax.jit
def kernel(vfeat, ffeat, aw1, ab1, aw2, ab2, aw3, ab3,
           bw1, bb1, bw2, bb2, bw3, bb3):
    atn_v, nv = _attention(vfeat, aw1, ab1, aw2, ab2, aw3, ab3)
    atn_f, nf = _attention(ffeat, bw1, bb1, bw2, bb2, bw3, bb3)
    new_vfeat, new_ffeat = _core(vfeat, ffeat, nv, nf, atn_v, atn_f)
    v_atn_out, _ = _attention(new_vfeat, aw1, ab1, aw2, ab2, aw3, ab3)
    f_atn_out, _ = _attention(new_ffeat, bw1, bb1, bw2, bb2, bw3, bb3)
    return (v_atn_out, new_vfeat, f_atn_out, new_ffeat)


# speedup vs baseline: 96.7300x; 1.0076x over previous
import functools

import jax
import jax.numpy as jnp
from jax.experimental import pallas as pl

_B = 4
_C = 1024
_T = 2048
_H = 512
_JBLK = 256
_K = _T // 8
_ACTION_T = 0.55
_BACKGROUND_T = 0.45
_SIM_T = 0.05
_BS_ITERS = 32


def _attn_kernel(lowp, x_ref, w1_ref, b1_ref, w2_ref, b2_ref, w3_ref, b3_ref,
                 atn_ref, nrm_ref):
    mmdt = jnp.bfloat16 if lowp else jnp.float32
    x = x_ref[0]
    nrm = jnp.sqrt(jnp.sum(x * x, axis=0, keepdims=True))
    nrm_ref[0] = jnp.maximum(nrm, 1e-12)

    def conv3(h, w_ref, b_ref):
        hm = h.astype(mmdt)
        y0 = jnp.dot(w_ref[0].astype(mmdt), hm,
                     preferred_element_type=jnp.float32)
        y1 = jnp.dot(w_ref[1].astype(mmdt), hm,
                     preferred_element_type=jnp.float32)
        y2 = jnp.dot(w_ref[2].astype(mmdt), hm,
                     preferred_element_type=jnp.float32)
        z = jnp.zeros((y0.shape[0], 1), jnp.float32)
        out = (y1
               + jnp.concatenate([z, y0[:, :-1]], axis=1)
               + jnp.concatenate([y2[:, 1:], z], axis=1)
               + b_ref[...])
        return jax.nn.leaky_relu(out, 0.2)

    h1 = conv3(x, w1_ref, b1_ref)
    h2 = conv3(h1, w2_ref, b2_ref)
    h3 = jnp.dot(w3_ref[...], h2, preferred_element_type=jnp.float32)
    atn_ref[0] = jax.nn.sigmoid(h3 + b3_ref[...])


def _attention(x, w1, b1, w2, b2, w3, b3, lowp=False):
    w1t = jnp.transpose(w1, (2, 0, 1))
    w2t = jnp.transpose(w2, (2, 0, 1))
    b1c = b1[:, None]
    b2c = b2[:, None]
    w3r = w3[:, :, 0]
    b3r = b3[None, :]
    full = lambda a: pl.BlockSpec(a.shape, lambda b: (0,) * a.ndim)
    atn, nrm = pl.pallas_call(
        functools.partial(_attn_kernel, lowp),
        grid=(_B,),
        in_specs=[
            pl.BlockSpec((1, _C, _T), lambda b: (b, 0, 0)),
            full(w1t), full(b1c), full(w2t), full(b2c), full(w3r), full(b3r),
        ],
        out_specs=(
            pl.BlockSpec((1, 1, _T), lambda b: (b, 0, 0)),
            pl.BlockSpec((1, 1, _T), lambda b: (b, 0, 0)),
        ),
        out_shape=(
            jax.ShapeDtypeStruct((_B, 1, _T), jnp.float32),
            jax.ShapeDtypeStruct((_B, 1, _T), jnp.float32),
        ),
    )(x, w1t, b1c, w2t, b2c, w3r, b3r)
    return atn, nrm


def _core_kernel(vf_ref, ff_ref, nv_ref, nf_ref, av_ref, af_ref,
                 outv_ref, outf_ref):
    j = pl.program_id(1)
    js = j * _JBLK
    vf = vf_ref[0]
    ff = ff_ref[0]
    vfj = vf_ref[0, :, pl.ds(js, _JBLK)]
    ffj = ff_ref[0, :, pl.ds(js, _JBLK)]

    cdims = (((0,), (0,)), ((), ()))
    vsim = jax.lax.dot_general(vfj, vf, cdims,
                               preferred_element_type=jnp.float32)
    fsim = jax.lax.dot_general(ffj, ff, cdims,
                               preferred_element_type=jnp.float32)
    inv_nv = 1.0 / nv_ref[0]
    inv_nf = 1.0 / nf_ref[0]
    inv_nv_c = jnp.reshape(1.0 / nv_ref[0, :, pl.ds(js, _JBLK)], (_JBLK, 1))
    inv_nf_c = jnp.reshape(1.0 / nf_ref[0, :, pl.ds(js, _JBLK)], (_JBLK, 1))
    fus = 0.5 * (vsim * inv_nv_c * inv_nv + fsim * inv_nf_c * inv_nf)
    fus = jnp.where(fus < _SIM_T, 0.0, fus)

    def bs_step(_, lohi):
        lo, hi = lohi
        mid = 0.5 * (lo + hi)
        cnt = jnp.sum((fus >= mid).astype(jnp.float32), axis=1,
                      keepdims=True)
        pred = cnt >= _K
        return jnp.where(pred, mid, lo), jnp.where(pred, hi, mid)

    lo, _ = jax.lax.fori_loop(
        0, _BS_ITERS, bs_step,
        (jnp.zeros((_JBLK, 1), jnp.float32),
         jnp.full((_JBLK, 1), 2.0, jnp.float32)))
    keep = (fus >= lo).astype(jnp.float32)

    av = av_ref[0]
    af = af_ref[0]
    act = ((av >= _ACTION_T) & (af >= _ACTION_T)).astype(jnp.float32)
    bg = ((av < _BACKGROUND_T) & (af < _BACKGROUND_T)).astype(jnp.float32)
    ab = act + bg
    avj = av_ref[0, :, pl.ds(js, _JBLK)]
    afj = af_ref[0, :, pl.ds(js, _JBLK)]
    act_c = jnp.reshape(
        ((avj >= _ACTION_T) & (afj >= _ACTION_T)).astype(jnp.float32),
        (_JBLK, 1))
    bg_c = jnp.reshape(
        ((avj < _BACKGROUND_T) & (afj < _BACKGROUND_T)).astype(jnp.float32),
        (_JBLK, 1))
    amb_c = 1.0 - act_c - bg_c
    gi = jax.lax.broadcasted_iota(jnp.int32, (_JBLK, _T), 1)
    gj = js + jax.lax.broadcasted_iota(jnp.int32, (_JBLK, _T), 0)
    eye = (gi == gj).astype(jnp.float32)
    m = act_c * act + bg_c * bg + amb_c * jnp.maximum(ab, eye)

    masked = fus * keep * m
    den = jnp.sum(masked, axis=1, keepdims=True)
    adj_t = masked * (1.0 / jnp.maximum(den, 1e-12))

    rdims = (((1,), (1,)), ((), ()))
    adj_bf = adj_t.astype(jnp.bfloat16)
    avg_v = jax.lax.dot_general(vf.astype(jnp.bfloat16), adj_bf, rdims,
                                preferred_element_type=jnp.float32)
    avg_f = jax.lax.dot_general(ff.astype(jnp.bfloat16), adj_bf, rdims,
                                preferred_element_type=jnp.float32)
    outv_ref[0] = (vfj + avg_v) * 0.5
    outf_ref[0] = (ffj + avg_f) * 0.5


def _core(vfeat, ffeat, nv, nf, av, af):
    nj = _T // _JBLK
    feat_spec = pl.BlockSpec((1, _C, _T), lambda b, j: (b, 0, 0))
    vec_spec = pl.BlockSpec((1, 1, _T), lambda b, j: (b, 0, 0))
    out_spec = pl.BlockSpec((1, _C, _JBLK), lambda b, j: (b, 0, j))
    return pl.pallas_call(
        _core_kernel,
        grid=(_B, nj),
        in_specs=[feat_spec, feat_spec, vec_spec, vec_spec, vec_spec,
                  vec_spec],
        out_specs=(out_spec, out_spec),
        out_shape=(
            jax.ShapeDtypeStruct((_B, _C, _T), jnp.float32),
            jax.ShapeDtypeStruct((_B, _C, _T), jnp.float32),
        ),
    )(vfeat, ffeat, nv, nf, av, af)


@jax.jit
def kernel(vfeat, ffeat, aw1, ab1, aw2, ab2, aw3, ab3,
           bw1, bb1, bw2, bb2, bw3, bb3):
    atn_v, nv = _attention(vfeat, aw1, ab1, aw2, ab2, aw3, ab3)
    atn_f, nf = _attention(ffeat, bw1, bb1, bw2, bb2, bw3, bb3)
    new_vfeat, new_ffeat = _core(vfeat, ffeat, nv, nf, atn_v, atn_f)
    v_atn_out, _ = _attention(new_vfeat, aw1, ab1, aw2, ab2, aw3, ab3,
                              lowp=True)
    f_atn_out, _ = _attention(new_ffeat, bw1, bb1, bw2, bb2, bw3, bb3,
                              lowp=True)
    return (v_atn_out, new_vfeat, f_atn_out, new_ffeat)


# speedup vs baseline: 159.4202x; 1.6481x over previous
import functools

import jax
import jax.numpy as jnp
from jax.experimental import pallas as pl

_B = 4
_C = 1024
_T = 2048
_H = 512
_JBLK = 256
_K = _T // 8
_ACTION_T = 0.55
_BACKGROUND_T = 0.45
_SIM_T = 0.05
_BS_ITERS = 32


def _attn_kernel(lowp, x_ref, w1_ref, b1_ref, w2_ref, b2_ref, w3_ref, b3_ref,
                 atn_ref, nrm_ref):
    mmdt = jnp.bfloat16 if lowp else jnp.float32
    x = x_ref[0]
    nrm = jnp.sqrt(jnp.sum(x * x, axis=0, keepdims=True))
    nrm_ref[0] = jnp.maximum(nrm, 1e-12)

    def conv3(h, w_ref, b_ref):
        hm = h.astype(mmdt)
        y0 = jnp.dot(w_ref[0].astype(mmdt), hm,
                     preferred_element_type=jnp.float32)
        y1 = jnp.dot(w_ref[1].astype(mmdt), hm,
                     preferred_element_type=jnp.float32)
        y2 = jnp.dot(w_ref[2].astype(mmdt), hm,
                     preferred_element_type=jnp.float32)
        z = jnp.zeros((y0.shape[0], 1), jnp.float32)
        out = (y1
               + jnp.concatenate([z, y0[:, :-1]], axis=1)
               + jnp.concatenate([y2[:, 1:], z], axis=1)
               + b_ref[...])
        return jax.nn.leaky_relu(out, 0.2)

    h1 = conv3(x, w1_ref, b1_ref)
    h2 = conv3(h1, w2_ref, b2_ref)
    h3 = jnp.dot(w3_ref[...], h2, preferred_element_type=jnp.float32)
    atn_ref[0] = jax.nn.sigmoid(h3 + b3_ref[...])


def _attention(x, w1, b1, w2, b2, w3, b3, lowp=False):
    w1t = jnp.transpose(w1, (2, 0, 1))
    w2t = jnp.transpose(w2, (2, 0, 1))
    b1c = b1[:, None]
    b2c = b2[:, None]
    w3r = w3[:, :, 0]
    b3r = b3[None, :]
    full = lambda a: pl.BlockSpec(a.shape, lambda b: (0,) * a.ndim)
    atn, nrm = pl.pallas_call(
        functools.partial(_attn_kernel, lowp),
        grid=(_B,),
        in_specs=[
            pl.BlockSpec((1, _C, _T), lambda b: (b, 0, 0)),
            full(w1t), full(b1c), full(w2t), full(b2c), full(w3r), full(b3r),
        ],
        out_specs=(
            pl.BlockSpec((1, 1, _T), lambda b: (b, 0, 0)),
            pl.BlockSpec((1, 1, _T), lambda b: (b, 0, 0)),
        ),
        out_shape=(
            jax.ShapeDtypeStruct((_B, 1, _T), jnp.float32),
            jax.ShapeDtypeStruct((_B, 1, _T), jnp.float32),
        ),
    )(x, w1t, b1c, w2t, b2c, w3r, b3r)
    return atn, nrm


def _core_kernel(vf_ref, ff_ref, nv_ref, nf_ref, av_ref, af_ref,
                 outv_ref, outf_ref):
    j = pl.program_id(1)
    js = j * _JBLK
    vf = vf_ref[0]
    ff = ff_ref[0]
    vfj = vf_ref[0, :, pl.ds(js, _JBLK)]
    ffj = ff_ref[0, :, pl.ds(js, _JBLK)]

    cdims = (((0,), (0,)), ((), ()))
    vsim = jax.lax.dot_general(vfj, vf, cdims,
                               preferred_element_type=jnp.float32)
    fsim = jax.lax.dot_general(ffj, ff, cdims,
                               preferred_element_type=jnp.float32)
    inv_nv = 1.0 / nv_ref[0]
    inv_nf = 1.0 / nf_ref[0]
    inv_nv_c = jnp.reshape(1.0 / nv_ref[0, :, pl.ds(js, _JBLK)], (_JBLK, 1))
    inv_nf_c = jnp.reshape(1.0 / nf_ref[0, :, pl.ds(js, _JBLK)], (_JBLK, 1))
    fus = 0.5 * (vsim * inv_nv_c * inv_nv + fsim * inv_nf_c * inv_nf)
    fus = jnp.where(fus < _SIM_T, 0.0, fus)

    def bs_step(_, lohi):
        lo, hi = lohi
        mid = 0.5 * (lo + hi)
        cnt = jnp.sum((fus >= mid).astype(jnp.float32), axis=1,
                      keepdims=True)
        pred = cnt >= _K
        return jnp.where(pred, mid, lo), jnp.where(pred, hi, mid)

    pos_cnt = jnp.sum((fus > 0.0).astype(jnp.float32), axis=1,
                      keepdims=True)
    need_search = jnp.any(pos_cnt >= _K)

    def full_search(_):
        lo, _ = jax.lax.fori_loop(
            0, _BS_ITERS, bs_step,
            (jnp.zeros((_JBLK, 1), jnp.float32),
             jnp.full((_JBLK, 1), 2.0, jnp.float32)))
        return lo

    lo = jax.lax.cond(need_search, full_search,
                      lambda _: jnp.zeros((_JBLK, 1), jnp.float32), 0)
    keep = (fus >= lo).astype(jnp.float32)

    av = av_ref[0]
    af = af_ref[0]
    act = ((av >= _ACTION_T) & (af >= _ACTION_T)).astype(jnp.float32)
    bg = ((av < _BACKGROUND_T) & (af < _BACKGROUND_T)).astype(jnp.float32)
    ab = act + bg
    avj = av_ref[0, :, pl.ds(js, _JBLK)]
    afj = af_ref[0, :, pl.ds(js, _JBLK)]
    act_c = jnp.reshape(
        ((avj >= _ACTION_T) & (afj >= _ACTION_T)).astype(jnp.float32),
        (_JBLK, 1))
    bg_c = jnp.reshape(
        ((avj < _BACKGROUND_T) & (afj < _BACKGROUND_T)).astype(jnp.float32),
        (_JBLK, 1))
    amb_c = 1.0 - act_c - bg_c
    gi = jax.lax.broadcasted_iota(jnp.int32, (_JBLK, _T), 1)
    gj = js + jax.lax.broadcasted_iota(jnp.int32, (_JBLK, _T), 0)
    eye = (gi == gj).astype(jnp.float32)
    m = act_c * act + bg_c * bg + amb_c * jnp.maximum(ab, eye)

    masked = fus * keep * m
    den = jnp.sum(masked, axis=1, keepdims=True)
    adj_t = masked * (1.0 / jnp.maximum(den, 1e-12))

    rdims = (((1,), (1,)), ((), ()))
    adj_bf = adj_t.astype(jnp.bfloat16)
    avg_v = jax.lax.dot_general(vf.astype(jnp.bfloat16), adj_bf, rdims,
                                preferred_element_type=jnp.float32)
    avg_f = jax.lax.dot_general(ff.astype(jnp.bfloat16), adj_bf, rdims,
                                preferred_element_type=jnp.float32)
    outv_ref[0] = (vfj + avg_v) * 0.5
    outf_ref[0] = (ffj + avg_f) * 0.5


def _core(vfeat, ffeat, nv, nf, av, af):
    nj = _T // _JBLK
    feat_spec = pl.BlockSpec((1, _C, _T), lambda b, j: (b, 0, 0))
    vec_spec = pl.BlockSpec((1, 1, _T), lambda b, j: (b, 0, 0))
    out_spec = pl.BlockSpec((1, _C, _JBLK), lambda b, j: (b, 0, j))
    return pl.pallas_call(
        _core_kernel,
        grid=(_B, nj),
        in_specs=[feat_spec, feat_spec, vec_spec, vec_spec, vec_spec,
                  vec_spec],
        out_specs=(out_spec, out_spec),
        out_shape=(
            jax.ShapeDtypeStruct((_B, _C, _T), jnp.float32),
            jax.ShapeDtypeStruct((_B, _C, _T), jnp.float32),
        ),
    )(vfeat, ffeat, nv, nf, av, af)


@jax.jit
def kernel(vfeat, ffeat, aw1, ab1, aw2, ab2, aw3, ab3,
           bw1, bb1, bw2, bb2, bw3, bb3):
    atn_v, nv = _attention(vfeat, aw1, ab1, aw2, ab2, aw3, ab3)
    atn_f, nf = _attention(ffeat, bw1, bb1, bw2, bb2, bw3, bb3)
    new_vfeat, new_ffeat = _core(vfeat, ffeat, nv, nf, atn_v, atn_f)
    v_atn_out, _ = _attention(new_vfeat, aw1, ab1, aw2, ab2, aw3, ab3,
                              lowp=True)
    f_atn_out, _ = _attention(new_ffeat, bw1, bb1, bw2, bb2, bw3, bb3,
                              lowp=True)
    return (v_atn_out, new_vfeat, f_atn_out, new_ffeat)
